# Initial kernel scaffold; baseline (speedup 1.0000x reference)
import functools

import jax
import jax.numpy as jnp
from jax import lax
from jax.experimental import pallas as pl
from jax.experimental.pallas import tpu as pltpu
from jax.experimental.pallas import tpu_sc as plsc

N = 10000
D = 256
DH = 128
E = 160000
NT = 16
CH = 128
NCHUNK = 80
EPT = CH * NCHUNK
EP = EPT * NT
ACC_ROWS = 10240
PAD_ROWS = ACC_ROWS - N
ROWS_OUT = N // NT
ZCH = 128
ZITER = ACC_ROWS // NT // ZCH


def _sc_body(compute_deg, *refs):
    if compute_deg:
        (xlo, xhi, srcp, dstp, mlo, mhi, deg_out,
         acc_sh, deg_sh, sidx, didx, rows, zrow, z16, ones16, sem) = refs
    else:
        (xlo, xhi, srcp, dstp, mlo, mhi,
         acc_sh, sidx, didx, rows, zrow, sem) = refs
        deg_out = deg_sh = z16 = ones16 = None

    c = lax.axis_index("c")
    s = lax.axis_index("s")

    def _zrow_init(i, _):
        r = i // (DH // 16)
        k = i % (DH // 16)
        zrow[r, pl.ds(k * 16, 16)] = jnp.zeros((16,), jnp.float32)
        return 0
    lax.fori_loop(0, ZCH * (DH // 16), _zrow_init, 0)

    if compute_deg:
        def _small_init(i, _):
            z16[i, :] = jnp.zeros((16,), jnp.float32)
            ones16[i, :] = jnp.ones((16,), jnp.float32)
            return 0
        lax.fori_loop(0, CH, _small_init, 0)

    for b in range(ZITER):
        r0 = (s * ZITER + b) * ZCH
        pltpu.sync_copy(zrow, acc_sh.at[pl.ds(r0, ZCH)])
    if compute_deg:
        @pl.when(c == 0)
        def _():
            for b in range(ZITER):
                r0 = (s * ZITER + b) * ZCH
                pltpu.sync_copy(z16, deg_sh.at[pl.ds(r0, ZCH)])

    plsc.subcore_barrier()

    def _edge_loop(x_hbm, with_deg):
        def step(k, _):
            base = pl.multiple_of((k * NT + s) * CH, CH)
            pltpu.sync_copy(srcp.at[pl.ds(base, CH)], sidx)
            pltpu.sync_copy(dstp.at[pl.ds(base, CH)], didx)
            pltpu.async_copy(x_hbm.at[sidx], rows, sem).wait()
            pltpu.sync_copy(rows, acc_sh.at[didx], add=True)
            if with_deg:
                pltpu.sync_copy(ones16, deg_sh.at[didx], add=True)
            return 0
        lax.fori_loop(0, NCHUNK, step, 0)

    @pl.when(c == 0)
    def _():
        _edge_loop(xlo, compute_deg)

    @pl.when(c == 1)
    def _():
        _edge_loop(xhi, False)

    plsc.subcore_barrier()

    ob = s * ROWS_OUT

    @pl.when(c == 0)
    def _():
        pltpu.sync_copy(acc_sh.at[pl.ds(ob, ROWS_OUT)], mlo.at[pl.ds(ob, ROWS_OUT)])
        if compute_deg:
            pltpu.sync_copy(deg_sh.at[pl.ds(ob, ROWS_OUT)], deg_out.at[pl.ds(ob, ROWS_OUT)])

    @pl.when(c == 1)
    def _():
        pltpu.sync_copy(acc_sh.at[pl.ds(ob, ROWS_OUT)], mhi.at[pl.ds(ob, ROWS_OUT)])


def _make_sc_agg(compute_deg):
    out_type = [
        jax.ShapeDtypeStruct((N, DH), jnp.float32),
        jax.ShapeDtypeStruct((N, DH), jnp.float32),
    ]
    scratch = [
        pltpu.VMEM_SHARED((ACC_ROWS, DH), jnp.float32),
    ]
    if compute_deg:
        out_type.append(jax.ShapeDtypeStruct((N, 16), jnp.float32))
        scratch.append(pltpu.VMEM_SHARED((ACC_ROWS, 16), jnp.float32))
    scratch += [
        pltpu.VMEM((CH,), jnp.int32),
        pltpu.VMEM((CH,), jnp.int32),
        pltpu.VMEM((CH, DH), jnp.float32),
        pltpu.VMEM((ZCH, DH), jnp.float32),
    ]
    if compute_deg:
        scratch += [
            pltpu.VMEM((CH, 16), jnp.float32),
            pltpu.VMEM((CH, 16), jnp.float32),
        ]
    scratch.append(pltpu.SemaphoreType.DMA)
    mesh = plsc.VectorSubcoreMesh(core_axis_name="c", subcore_axis_name="s")
    return pl.kernel(
        functools.partial(_sc_body, compute_deg),
        out_type=out_type,
        mesh=mesh,
        scratch_types=scratch,
    )


_sc_agg_deg = _make_sc_agg(True)
_sc_agg = _make_sc_agg(False)


def _tc_body(final, mlo, mhi, deg16, hlo, hhi, wl, bl, wr, g, beta, *outs):
    mw = (jnp.dot(mlo[...], wl[:DH, :], preferred_element_type=jnp.float32)
          + jnp.dot(mhi[...], wl[DH:, :], preferred_element_type=jnp.float32))
    hw = (jnp.dot(hlo[...], wr[:DH, :], preferred_element_type=jnp.float32)
          + jnp.dot(hhi[...], wr[DH:, :], preferred_element_type=jnp.float32))
    d = jnp.maximum(deg16[:, :1], 1.0)
    t = mw / d + bl[...] + hw
    if final:
        m = jnp.max(t, axis=1, keepdims=True)
        lse = jnp.log(jnp.sum(jnp.exp(t - m), axis=1, keepdims=True)) + m
        outs[0][...] = t - lse
    else:
        mu = jnp.mean(t, axis=0, keepdims=True)
        var = jnp.mean((t - mu) ** 2, axis=0, keepdims=True)
        h = jnp.maximum((t - mu) * lax.rsqrt(var + 1e-5) * g[...] + beta[...], 0.0)
        outs[0][...] = h[:, :DH]
        outs[1][...] = h[:, DH:]


def _tc_layer(final):
    if final:
        out_shape = [jax.ShapeDtypeStruct((N, D), jnp.float32)]
    else:
        out_shape = [jax.ShapeDtypeStruct((N, DH), jnp.float32),
                     jax.ShapeDtypeStruct((N, DH), jnp.float32)]
    return pl.pallas_call(
        functools.partial(_tc_body, final),
        out_shape=out_shape,
    )


_tc_bn_relu = _tc_layer(False)
_tc_final = _tc_layer(True)


def kernel(x, edge_index, Wl0, bl0, Wr0, g0, beta0,
           Wl1, bl1, Wr1, g1, beta1, Wl2, bl2, Wr2):
    src, dst = edge_index[0], edge_index[1]
    ar = jnp.arange(EP - E, dtype=jnp.int32)
    srcp = jnp.concatenate([src, ar % 128])
    dstp = jnp.concatenate([dst, N + (ar % PAD_ROWS)])
    xlo, xhi = x[:, :DH], x[:, DH:]

    mlo, mhi, deg16 = _sc_agg_deg(xlo, xhi, srcp, dstp)
    hlo, hhi = _tc_bn_relu(mlo, mhi, deg16, xlo, xhi, Wl0, bl0.reshape(1, D),
                           Wr0, g0.reshape(1, D), beta0.reshape(1, D))
    mlo, mhi = _sc_agg(hlo, hhi, srcp, dstp)
    hlo, hhi = _tc_bn_relu(mlo, mhi, deg16, hlo, hhi, Wl1, bl1.reshape(1, D),
                           Wr1, g1.reshape(1, D), beta1.reshape(1, D))
    mlo, mhi = _sc_agg(hlo, hhi, srcp, dstp)
    (out,) = _tc_final(mlo, mhi, deg16, hlo, hhi, Wl2, bl2.reshape(1, D),
                       Wr2, jnp.zeros((1, D), jnp.float32), jnp.zeros((1, D), jnp.float32))
    return out


# speedup vs baseline: 3.1639x; 3.1639x over previous
import functools

import jax
import jax.numpy as jnp
from jax import lax
from jax.experimental import pallas as pl
from jax.experimental.pallas import tpu as pltpu
from jax.experimental.pallas import tpu_sc as plsc

N = 10000
D = 256
DH = 128
E = 160000
NT = 16
CH = 128
NCHUNK = 80
EPT = CH * NCHUNK
EP = EPT * NT
ACC_ROWS = 10240
PAD_ROWS = ACC_ROWS - N
ROWS_OUT = ACC_ROWS // NT
ZCH = 128
ZITER = ACC_ROWS // NT // ZCH


DEGR = ACC_ROWS * 16 // 128


def _deg_body(dstp, deg_out, deg_sh, didx, z16, ones16, t16, t128):
    c = lax.axis_index("c")
    s = lax.axis_index("s")

    @pl.when(c == 0)
    def _():
        def _small_init(i, _):
            z16[i, :] = jnp.zeros((16,), jnp.float32)
            ones16[i, :] = jnp.ones((16,), jnp.float32)
            return 0
        lax.fori_loop(0, CH, _small_init, 0)
        for b in range(ZITER):
            r0 = (s * ZITER + b) * ZCH
            pltpu.sync_copy(z16, deg_sh.at[pl.ds(r0, ZCH)])

    plsc.subcore_barrier()

    @pl.when(c == 0)
    def _():
        def step(k, _):
            base = pl.multiple_of((k * NT + s) * CH, CH)
            pltpu.sync_copy(dstp.at[pl.ds(base, CH)], didx)
            pltpu.sync_copy(ones16, deg_sh.at[didx], add=True)
            return 0
        lax.fori_loop(0, NCHUNK, step, 0)

    plsc.subcore_barrier()

    @pl.when(c == 0)
    def _():
        for b in range(ZITER):
            pltpu.sync_copy(deg_sh.at[pl.ds((s * ZITER + b) * ZCH, ZCH)], t16)

            def repack(j, _):
                t128[j // 8, pl.ds((j % 8) * 16, 16)] = t16[j, :]
                return 0
            lax.fori_loop(0, ZCH, repack, 0)
            ob = pl.multiple_of((s * ZITER + b) * (ZCH // 8), 8)
            pltpu.sync_copy(t128, deg_out.at[pl.ds(ob, ZCH // 8)])


@functools.lru_cache(maxsize=None)
def _make_deg():
    mesh = plsc.VectorSubcoreMesh(core_axis_name="c", subcore_axis_name="s")
    return pl.kernel(
        _deg_body,
        out_type=[jax.ShapeDtypeStruct((DEGR, 128), jnp.float32)],
        mesh=mesh,
        scratch_types=[
            pltpu.VMEM_SHARED((ACC_ROWS, 16), jnp.float32),
            pltpu.VMEM((CH,), jnp.int32),
            pltpu.VMEM((CH, 16), jnp.float32),
            pltpu.VMEM((CH, 16), jnp.float32),
            pltpu.VMEM((ZCH, 16), jnp.float32),
            pltpu.VMEM((ZCH // 8, 128), jnp.float32),
        ],
    )


def _sc_body(*refs):
    (xlo, xhi, srcp, dstp, mlo, mhi,
     acc_sh, sidx, didx, rows, sem) = refs

    c = lax.axis_index("c")
    s = lax.axis_index("s")

    def _zrow_init(i, _):
        r = i // (DH // 16)
        k = i % (DH // 16)
        rows[r, pl.ds(k * 16, 16)] = jnp.zeros((16,), jnp.float32)
        return 0
    lax.fori_loop(0, ZCH * (DH // 16), _zrow_init, 0)

    for b in range(ZITER):
        r0 = (s * ZITER + b) * ZCH
        pltpu.sync_copy(rows, acc_sh.at[pl.ds(r0, ZCH)])

    plsc.subcore_barrier()

    def _edge_loop(x_hbm):
        def step(k, _):
            base = pl.multiple_of((k * NT + s) * CH, CH)
            pltpu.sync_copy(srcp.at[pl.ds(base, CH)], sidx)
            pltpu.sync_copy(dstp.at[pl.ds(base, CH)], didx)
            pltpu.async_copy(x_hbm.at[sidx], rows, sem).wait()
            pltpu.sync_copy(rows, acc_sh.at[didx], add=True)
            return 0
        lax.fori_loop(0, NCHUNK, step, 0)

    @pl.when(c == 0)
    def _():
        _edge_loop(xlo)

    @pl.when(c == 1)
    def _():
        _edge_loop(xhi)

    plsc.subcore_barrier()

    ob = pl.multiple_of(s * ROWS_OUT, 8)

    @pl.when(c == 0)
    def _():
        pltpu.sync_copy(acc_sh.at[pl.ds(ob, ROWS_OUT)], mlo.at[pl.ds(ob, ROWS_OUT)])

    @pl.when(c == 1)
    def _():
        pltpu.sync_copy(acc_sh.at[pl.ds(ob, ROWS_OUT)], mhi.at[pl.ds(ob, ROWS_OUT)])


@functools.lru_cache(maxsize=None)
def _make_sc_agg():
    mesh = plsc.VectorSubcoreMesh(core_axis_name="c", subcore_axis_name="s")
    return pl.kernel(
        _sc_body,
        out_type=[
            jax.ShapeDtypeStruct((ACC_ROWS, DH), jnp.float32),
            jax.ShapeDtypeStruct((ACC_ROWS, DH), jnp.float32),
        ],
        mesh=mesh,
        scratch_types=[
            pltpu.VMEM_SHARED((ACC_ROWS, DH), jnp.float32),
            pltpu.VMEM((CH,), jnp.int32),
            pltpu.VMEM((CH,), jnp.int32),
            pltpu.VMEM((CH, DH), jnp.float32),
            pltpu.SemaphoreType.DMA,
        ],
    )


def _tc_body(final, mlo, mhi, deg16, hlo, hhi, wl, bl, wr, g, beta, *outs):
    mw = (jnp.dot(mlo[:N], wl[:DH, :], preferred_element_type=jnp.float32)
          + jnp.dot(mhi[:N], wl[DH:, :], preferred_element_type=jnp.float32))
    hw = (jnp.dot(hlo[...], wr[:DH, :], preferred_element_type=jnp.float32)
          + jnp.dot(hhi[...], wr[DH:, :], preferred_element_type=jnp.float32))
    d = jnp.maximum(deg16[:N, :1], 1.0)
    t = mw / d + bl[...] + hw
    if final:
        m = jnp.max(t, axis=1, keepdims=True)
        lse = jnp.log(jnp.sum(jnp.exp(t - m), axis=1, keepdims=True)) + m
        outs[0][...] = t - lse
    else:
        mu = jnp.mean(t, axis=0, keepdims=True)
        var = jnp.mean((t - mu) ** 2, axis=0, keepdims=True)
        h = jnp.maximum((t - mu) * lax.rsqrt(var + 1e-5) * g[...] + beta[...], 0.0)
        outs[0][...] = h[:, :DH]
        outs[1][...] = h[:, DH:]


def _tc_layer(final):
    if final:
        out_shape = [jax.ShapeDtypeStruct((N, D), jnp.float32)]
    else:
        out_shape = [jax.ShapeDtypeStruct((N, DH), jnp.float32),
                     jax.ShapeDtypeStruct((N, DH), jnp.float32)]
    return pl.pallas_call(
        functools.partial(_tc_body, final),
        out_shape=out_shape,
    )


_tc_bn_relu = _tc_layer(False)
_tc_final = _tc_layer(True)


def kernel(x, edge_index, Wl0, bl0, Wr0, g0, beta0,
           Wl1, bl1, Wr1, g1, beta1, Wl2, bl2, Wr2):
    src, dst = edge_index[0], edge_index[1]
    ar = jnp.arange(EP - E, dtype=jnp.int32)
    srcp = jnp.concatenate([src, ar % 128])
    dstp = jnp.concatenate([dst, N + (ar % PAD_ROWS)])
    xlo, xhi = x[:, :DH], x[:, DH:]

    ones_x = jnp.ones((N, DH), jnp.float32)
    deg16, _ = _make_sc_agg()(ones_x, ones_x, srcp, dstp)
    mlo, mhi = _make_sc_agg()(xlo, xhi, srcp, dstp)
    hlo, hhi = _tc_bn_relu(mlo, mhi, deg16, xlo, xhi, Wl0, bl0.reshape(1, D),
                           Wr0, g0.reshape(1, D), beta0.reshape(1, D))
    mlo, mhi = _make_sc_agg()(hlo, hhi, srcp, dstp)
    hlo, hhi = _tc_bn_relu(mlo, mhi, deg16, hlo, hhi, Wl1, bl1.reshape(1, D),
                           Wr1, g1.reshape(1, D), beta1.reshape(1, D))
    mlo, mhi = _make_sc_agg()(hlo, hhi, srcp, dstp)
    (out,) = _tc_final(mlo, mhi, deg16, hlo, hhi, Wl2, bl2.reshape(1, D),
                       Wr2, jnp.zeros((1, D), jnp.float32), jnp.zeros((1, D), jnp.float32))
    return out


# speedup vs baseline: 6.7631x; 2.1376x over previous
import functools

import jax
import jax.numpy as jnp
from jax import lax
from jax.experimental import pallas as pl
from jax.experimental.pallas import tpu as pltpu
from jax.experimental.pallas import tpu_sc as plsc

N = 10000
D = 256
DH = 128
E = 160000
NT = 16
CH = 128
NCHUNK = 80
GRP = 8
NGRP = NCHUNK // GRP
EPT = CH * NCHUNK
EP = EPT * NT
ACC_ROWS = 10240
PAD_ROWS = ACC_ROWS - N
ROWS_OUT = ACC_ROWS // NT
ZCH = 128
ZITER = ACC_ROWS // NT // ZCH


def _zero_buf(buf, nrow):
    def body(i, _):
        buf[i // 8, pl.ds((i % 8) * 16, 16)] = jnp.zeros((16,), jnp.float32)
        return 0
    lax.fori_loop(0, nrow * 8, body, 0)


def _add_src_offset(sd, coff):
    def body(i, _):
        r = i // 8
        v = i % 8
        sl = pl.ds(v * 16, 16)
        sd[0, r, sl] = sd[0, r, sl] + coff
        return 0
    lax.fori_loop(0, GRP * 8, body, 0)


def _sc_body(xs2, sdp, m2, acc_sh, sd0, sd1, rows0, rows1, sem0, sem1):
    c = lax.axis_index("c")
    s = lax.axis_index("s")
    coff = c * N

    _zero_buf(rows0, CH)
    for b in range(ZITER):
        r0 = (s * ZITER + b) * ZCH
        pltpu.sync_copy(rows0, acc_sh.at[pl.ds(r0, ZCH)])

    plsc.subcore_barrier()

    pltpu.sync_copy(sdp.at[s, 0], sd0)
    _add_src_offset(sd0, coff)
    pltpu.async_copy(xs2.at[sd0.at[0, 0]], rows0, sem0)

    def group_body(g, sd_cur, sd_nxt):
        @pl.when(g + 1 < NGRP)
        def _():
            pltpu.sync_copy(sdp.at[s, g + 1], sd_nxt)
            _add_src_offset(sd_nxt, coff)
        for j in range(GRP):
            if j % 2 == 0:
                r_cur, r_nxt, s_cur, s_nxt = rows0, rows1, sem0, sem1
            else:
                r_cur, r_nxt, s_cur, s_nxt = rows1, rows0, sem1, sem0
            if j < GRP - 1:
                pltpu.async_copy(xs2.at[sd_cur.at[0, j + 1]], r_nxt, s_nxt)
            else:
                @pl.when(g + 1 < NGRP)
                def _():
                    pltpu.async_copy(xs2.at[sd_nxt.at[0, 0]], r_nxt, s_nxt)
            pltpu.make_async_copy(xs2.at[sd_cur.at[0, j]], r_cur, s_cur).wait()
            pltpu.sync_copy(r_cur, acc_sh.at[sd_cur.at[1, j]], add=True)

    def gloop(g, _):
        @pl.when(lax.rem(g, 2) == 0)
        def _():
            group_body(g, sd0, sd1)

        @pl.when(lax.rem(g, 2) == 1)
        def _():
            group_body(g, sd1, sd0)
        return 0
    lax.fori_loop(0, NGRP, gloop, 0)

    plsc.subcore_barrier()

    ob = pl.multiple_of(c * ACC_ROWS + s * ROWS_OUT, 8)
    pltpu.sync_copy(acc_sh.at[pl.ds(s * ROWS_OUT, ROWS_OUT)],
                    m2.at[pl.ds(ob, ROWS_OUT)])


@functools.lru_cache(maxsize=None)
def _make_sc_agg():
    mesh = plsc.VectorSubcoreMesh(core_axis_name="c", subcore_axis_name="s")
    return pl.kernel(
        _sc_body,
        out_type=[jax.ShapeDtypeStruct((2 * ACC_ROWS, DH), jnp.float32)],
        mesh=mesh,
        scratch_types=[
            pltpu.VMEM_SHARED((ACC_ROWS, DH), jnp.float32),
            pltpu.VMEM((2, GRP, CH), jnp.int32),
            pltpu.VMEM((2, GRP, CH), jnp.int32),
            pltpu.VMEM((CH, DH), jnp.float32),
            pltpu.VMEM((CH, DH), jnp.float32),
            pltpu.SemaphoreType.DMA,
            pltpu.SemaphoreType.DMA,
        ],
    )


def _deg_body(sdp, dg2, deg_sh, sd0, sd1, ones128):
    c = lax.axis_index("c")
    s = lax.axis_index("s")

    _zero_buf(ones128, CH)
    for b in range(ZITER):
        r0 = (s * ZITER + b) * ZCH
        pltpu.sync_copy(ones128, deg_sh.at[pl.ds(r0, ZCH)])

    def fill_ones(i, _):
        ones128[i // 8, pl.ds((i % 8) * 16, 16)] = jnp.ones((16,), jnp.float32)
        return 0
    lax.fori_loop(0, CH * 8, fill_ones, 0)

    plsc.subcore_barrier()

    g0 = c * (NGRP // 2)
    pltpu.sync_copy(sdp.at[s, g0], sd0)

    def group_body(g, sd_cur, sd_nxt):
        @pl.when(g + 1 < NGRP // 2)
        def _():
            pltpu.sync_copy(sdp.at[s, g0 + g + 1], sd_nxt)
        for j in range(GRP):
            pltpu.sync_copy(ones128, deg_sh.at[sd_cur.at[1, j]], add=True)

    def gloop(g, _):
        @pl.when(lax.rem(g, 2) == 0)
        def _():
            group_body(g, sd0, sd1)

        @pl.when(lax.rem(g, 2) == 1)
        def _():
            group_body(g, sd1, sd0)
        return 0
    lax.fori_loop(0, NGRP // 2, gloop, 0)

    plsc.subcore_barrier()

    ob = pl.multiple_of(c * ACC_ROWS + s * ROWS_OUT, 8)
    pltpu.sync_copy(deg_sh.at[pl.ds(s * ROWS_OUT, ROWS_OUT)],
                    dg2.at[pl.ds(ob, ROWS_OUT)])


@functools.lru_cache(maxsize=None)
def _make_deg():
    mesh = plsc.VectorSubcoreMesh(core_axis_name="c", subcore_axis_name="s")
    return pl.kernel(
        _deg_body,
        out_type=[jax.ShapeDtypeStruct((2 * ACC_ROWS, DH), jnp.float32)],
        mesh=mesh,
        scratch_types=[
            pltpu.VMEM_SHARED((ACC_ROWS, DH), jnp.float32),
            pltpu.VMEM((2, GRP, CH), jnp.int32),
            pltpu.VMEM((2, GRP, CH), jnp.int32),
            pltpu.VMEM((CH, DH), jnp.float32),
        ],
    )


def _tc_bn_body(m2, dg2, hs2, wl, bl, wr, g, beta, out):
    mw = (jnp.dot(m2[:N], wl[:DH, :], preferred_element_type=jnp.float32)
          + jnp.dot(m2[ACC_ROWS:ACC_ROWS + N], wl[DH:, :],
                    preferred_element_type=jnp.float32))
    hw = (jnp.dot(hs2[:N], wr[:DH, :], preferred_element_type=jnp.float32)
          + jnp.dot(hs2[N:], wr[DH:, :], preferred_element_type=jnp.float32))
    d = jnp.maximum(dg2[:N, :1] + dg2[ACC_ROWS:ACC_ROWS + N, :1], 1.0)
    t = mw / d + bl[...] + hw
    mu = jnp.mean(t, axis=0, keepdims=True)
    var = jnp.mean((t - mu) ** 2, axis=0, keepdims=True)
    h = jnp.maximum((t - mu) * lax.rsqrt(var + 1e-5) * g[...] + beta[...], 0.0)
    out[:N] = h[:, :DH]
    out[N:] = h[:, DH:]


_tc_bn_relu = pl.pallas_call(
    _tc_bn_body,
    out_shape=[jax.ShapeDtypeStruct((2 * N, DH), jnp.float32)],
)

BF = 2000


def _tc_final_body(mlo, mhi, dga, dgb, hlo, hhi, wl, bl, wr, out):
    mw = (jnp.dot(mlo[...], wl[:DH, :], preferred_element_type=jnp.float32)
          + jnp.dot(mhi[...], wl[DH:, :], preferred_element_type=jnp.float32))
    hw = (jnp.dot(hlo[...], wr[:DH, :], preferred_element_type=jnp.float32)
          + jnp.dot(hhi[...], wr[DH:, :], preferred_element_type=jnp.float32))
    d = jnp.maximum(dga[:, :1] + dgb[:, :1], 1.0)
    t = mw / d + bl[...] + hw
    m = jnp.max(t, axis=1, keepdims=True)
    lse = jnp.log(jnp.sum(jnp.exp(t - m), axis=1, keepdims=True)) + m
    out[...] = t - lse


def _blk(i):
    return (i, 0)


def _rep(i):
    return (0, 0)


_tc_final = pl.pallas_call(
    _tc_final_body,
    grid=(N // BF,),
    in_specs=[pl.BlockSpec((BF, DH), _blk)] * 6
    + [pl.BlockSpec((D, D), _rep), pl.BlockSpec((1, D), _rep),
       pl.BlockSpec((D, D), _rep)],
    out_specs=pl.BlockSpec((BF, D), _blk),
    out_shape=jax.ShapeDtypeStruct((N, D), jnp.float32),
)


def kernel(x, edge_index, Wl0, bl0, Wr0, g0, beta0,
           Wl1, bl1, Wr1, g1, beta1, Wl2, bl2, Wr2):
    src, dst = edge_index[0], edge_index[1]
    ar = jnp.arange(EP - E, dtype=jnp.int32)
    srcp = jnp.concatenate([src, ar % 128])
    dstp = jnp.concatenate([dst, N + (ar % PAD_ROWS)])
    srcc = srcp.reshape(NCHUNK, NT, CH).transpose(1, 0, 2).reshape(NT, NGRP, GRP, CH)
    dstc = dstp.reshape(NCHUNK, NT, CH).transpose(1, 0, 2).reshape(NT, NGRP, GRP, CH)
    sdp = jnp.stack([srcc, dstc], axis=2)
    xs2 = jnp.concatenate([x[:, :DH], x[:, DH:]], axis=0)

    (dg2,) = _make_deg()(sdp)
    (m2,) = _make_sc_agg()(xs2, sdp)
    (hs2,) = _tc_bn_relu(m2, dg2, xs2, Wl0, bl0.reshape(1, D),
                         Wr0, g0.reshape(1, D), beta0.reshape(1, D))
    (m2,) = _make_sc_agg()(hs2, sdp)
    (hs2,) = _tc_bn_relu(m2, dg2, hs2, Wl1, bl1.reshape(1, D),
                         Wr1, g1.reshape(1, D), beta1.reshape(1, D))
    (m2,) = _make_sc_agg()(hs2, sdp)
    out = _tc_final(m2[:N], m2[ACC_ROWS:ACC_ROWS + N],
                    dg2[:N], dg2[ACC_ROWS:ACC_ROWS + N],
                    hs2[:N], hs2[N:], Wl2, bl2.reshape(1, D), Wr2)
    return out
